# Initial kernel scaffold; baseline (speedup 1.0000x reference)
import functools

import jax
import jax.numpy as jnp
from jax import lax
from jax.experimental import pallas as pl
from jax.experimental.pallas import tpu as pltpu
from jax.experimental.pallas import tpu_sc as plsc

NUM_HEADS = 16
HEAD_DIM = 16
HIDDEN = NUM_HEADS * HEAD_DIM
SCALE = float(HEAD_DIM) ** 0.5
NODES = 10000
EDGES = 160000

NC = 2
NS = 16
HH = NUM_HEADS // NC
HW = HH * HEAD_DIM
CHUNK = 128
E_PAD = 163840
N_CHUNKS = E_PAD // (NS * CHUNK)
ACC_W = HW + 16
ACC_ROWS = 10240
NB_CHUNK = 125
NB_PER_SUB = 5


def _sc_body(ktab, qtab, vtab, srcg, dstg, dsts, out,
             acc, kbuf, qbuf, vbuf, msg, sidx, didx, scidx, nbuf, obuf, sem):
    c = lax.axis_index("c")
    s = lax.axis_index("s")
    zero16 = jnp.zeros((16,), jnp.float32)
    iota16 = lax.iota(jnp.int32, 16)

    @pl.loop(0, CHUNK)
    def _zero_rows(r):
        for cb in range(ACC_W // 16):
            msg[r, pl.ds(cb * 16, 16)] = zero16

    @pl.loop(0, ACC_ROWS // (NS * CHUNK))
    def _zero_acc(m):
        base = s * (ACC_ROWS // NS) + m * CHUNK
        pltpu.sync_copy(msg, acc.at[pl.ds(base, CHUNK)])

    plsc.subcore_barrier()

    @pl.loop(0, N_CHUNKS)
    def _chunk(j):
        pltpu.sync_copy(srcg.at[c, s, j], sidx)
        pltpu.sync_copy(dstg.at[c, s, j], didx)
        pltpu.sync_copy(dsts.at[s, j], scidx)
        cp_k = pltpu.async_copy(ktab.at[sidx], kbuf, sem)
        cp_q = pltpu.async_copy(qtab.at[didx], qbuf, sem)
        cp_v = pltpu.async_copy(vtab.at[sidx], vbuf, sem)
        cp_k.wait()
        cp_q.wait()
        cp_v.wait()

        @pl.loop(0, CHUNK // 16)
        def _group(g):
            rows = iota16 + g * 16
            for h in range(HH):
                dot = zero16
                for d in range(HEAD_DIM):
                    col = jnp.full((16,), h * HEAD_DIM + d, jnp.int32)
                    kv = plsc.load_gather(kbuf, [rows, col])
                    qv = plsc.load_gather(qbuf, [rows, col])
                    dot = dot + kv * qv
                sc = dot * (1.0 / SCALE)
                sc = jnp.minimum(jnp.maximum(sc, -5.0), 5.0)
                es = jnp.exp(sc)
                zcol = jnp.full((16,), HW + h, jnp.int32)
                plsc.store_scatter(msg, [rows, zcol], es)
                for d in range(HEAD_DIM):
                    col = jnp.full((16,), h * HEAD_DIM + d, jnp.int32)
                    vv = plsc.load_gather(vbuf, [rows, col])
                    plsc.store_scatter(msg, [rows, col], vv * es)

        pltpu.sync_copy(msg, acc.at[scidx], add=True)

    plsc.subcore_barrier()

    @pl.loop(0, NB_PER_SUB)
    def _norm(m):
        base = s * (NB_PER_SUB * NB_CHUNK) + m * NB_CHUNK
        pltpu.sync_copy(acc.at[pl.ds(base, NB_CHUNK)], nbuf)

        @pl.loop(0, NB_CHUNK)
        def _node(n):
            nvec = jnp.full((16,), n, jnp.int32)
            for h in range(HH):
                zcol = jnp.full((16,), HW + h, jnp.int32)
                zh = plsc.load_gather(nbuf, [nvec, zcol])
                wv = nbuf[n, pl.ds(h * HEAD_DIM, 16)]
                obuf[n, pl.ds(h * HEAD_DIM, 16)] = wv / (zh + 1e-6)

        pltpu.sync_copy(obuf, out.at[c, pl.ds(base, NB_CHUNK)])


@jax.jit
def _run(ktab, qtab, vtab, srcg, dstg, dsts):
    mesh = plsc.VectorSubcoreMesh(core_axis_name="c", subcore_axis_name="s",
                                  num_cores=NC, num_subcores=NS)
    return pl.kernel(
        _sc_body,
        out_type=jax.ShapeDtypeStruct((NC, NODES, HW), jnp.float32),
        mesh=mesh,
        scratch_types=[
            pltpu.VMEM_SHARED((ACC_ROWS, ACC_W), jnp.float32),
            pltpu.VMEM((CHUNK, HW), jnp.float32),
            pltpu.VMEM((CHUNK, HW), jnp.float32),
            pltpu.VMEM((CHUNK, HW), jnp.float32),
            pltpu.VMEM((CHUNK, ACC_W), jnp.float32),
            pltpu.VMEM((CHUNK,), jnp.int32),
            pltpu.VMEM((CHUNK,), jnp.int32),
            pltpu.VMEM((CHUNK,), jnp.int32),
            pltpu.VMEM((NB_CHUNK, ACC_W), jnp.float32),
            pltpu.VMEM((NB_CHUNK, HW), jnp.float32),
            pltpu.SemaphoreType.DMA,
        ],
    )(ktab, qtab, vtab, srcg, dstg, dsts)


def kernel(q, k, v, edge_index):
    batch, node_num = q.shape[0], q.shape[1]

    def half_tab(x):
        return (x.reshape(NODES, NC, HW)
                 .transpose(1, 0, 2)
                 .reshape(NC * NODES, HW))

    ktab = half_tab(k)
    qtab = half_tab(q)
    vtab = half_tab(v)

    src = edge_index[0].astype(jnp.int32)
    dst = edge_index[1].astype(jnp.int32)
    pad = E_PAD - EDGES
    src_p = jnp.concatenate([src, jnp.zeros((pad,), jnp.int32)])
    dst_gp = jnp.concatenate([dst, jnp.zeros((pad,), jnp.int32)])
    dst_sp = jnp.concatenate([dst, jnp.full((pad,), NODES, jnp.int32)])
    srcg = jnp.stack([src_p, src_p + NODES]).reshape(NC, NS, N_CHUNKS, CHUNK)
    dstg = jnp.stack([dst_gp, dst_gp + NODES]).reshape(NC, NS, N_CHUNKS, CHUNK)
    dsts = dst_sp.reshape(NS, N_CHUNKS, CHUNK)

    out2 = _run(ktab, qtab, vtab, srcg, dstg, dsts)
    return out2.transpose(1, 0, 2).reshape(batch, node_num, HIDDEN)


# speedup vs baseline: 6.1284x; 6.1284x over previous
import jax
import jax.numpy as jnp
from jax import lax
from jax.experimental import pallas as pl
from jax.experimental.pallas import tpu as pltpu
from jax.experimental.pallas import tpu_sc as plsc

NUM_HEADS = 16
HEAD_DIM = 16
HIDDEN = NUM_HEADS * HEAD_DIM
SCALE = float(HEAD_DIM) ** 0.5
NODES = 10000
EDGES = 160000

NC = 2
NS = 16
HH = NUM_HEADS // NC
HW = HH * HEAD_DIM
CHUNK = 64
E_PAD = 163840
N_CHUNKS = E_PAD // (NS * CHUNK)
WV_ROWS = 10240
ZB = WV_ROWS
Z_ROWS = 640
ACC_ROWS = WV_ROWS + Z_ROWS
NB_CHUNK = 64
NB_PER_SUB = 10


def _sc_body(ktab, qtab, vtab, srcg, dstg, dsts, out,
             acc, kbuf, qbuf, msg, zmsg, sidx, didx, scidx, zsc,
             zstage, sem):
    c = lax.axis_index("c")
    s = lax.axis_index("s")
    zero16 = jnp.zeros((16,), jnp.float32)
    iota16 = lax.iota(jnp.int32, 16)

    @pl.loop(0, CHUNK)
    def _zero_rows(r):
        for cb in range(HW // 16):
            msg[r, pl.ds(cb * 16, 16)] = zero16
            zmsg[r, pl.ds(cb * 16, 16)] = zero16

    @pl.loop(0, 11)
    def _zero_acc(m):
        t = m * NS + s
        @pl.when(t < ACC_ROWS // CHUNK)
        def _():
            pltpu.sync_copy(msg, acc.at[pl.ds(t * CHUNK, CHUNK)])

    plsc.subcore_barrier()

    @pl.loop(0, N_CHUNKS)
    def _chunk(j):
        pltpu.sync_copy(srcg.at[c, s, j], sidx)
        pltpu.sync_copy(dstg.at[c, s, j], didx)
        pltpu.sync_copy(dsts.at[s, j], scidx)
        cp_k = pltpu.async_copy(ktab.at[sidx], kbuf, sem)
        cp_q = pltpu.async_copy(qtab.at[didx], qbuf, sem)
        cp_v = pltpu.async_copy(vtab.at[sidx], msg, sem)
        for g in range(CHUNK // 16):
            dv = scidx[pl.ds(g * 16, 16)]
            zsc[pl.ds(g * 16, 16)] = ZB + lax.shift_right_logical(dv, 4)
        cp_k.wait()
        cp_q.wait()
        cp_v.wait()

        @pl.loop(0, CHUNK // 16)
        def _group(g):
            rows = iota16 + g * 16
            dv = scidx[pl.ds(g * 16, 16)]
            zc0 = lax.shift_left(jnp.bitwise_and(dv, 15), 3)
            for h in range(HH):
                dot = zero16
                for d in range(HEAD_DIM):
                    col = jnp.full((16,), h * HEAD_DIM + d, jnp.int32)
                    kv = plsc.load_gather(kbuf, [rows, col])
                    qv = plsc.load_gather(qbuf, [rows, col])
                    dot = dot + kv * qv
                sc = dot * (1.0 / SCALE)
                sc = jnp.minimum(jnp.maximum(sc, -5.0), 5.0)
                es = jnp.exp(sc)
                plsc.store_scatter(zmsg, [rows, zc0 + h], es)
                for d in range(HEAD_DIM):
                    col = jnp.full((16,), h * HEAD_DIM + d, jnp.int32)
                    vv = plsc.load_gather(msg, [rows, col])
                    plsc.store_scatter(msg, [rows, col], vv * es)

        pltpu.sync_copy(msg, acc.at[scidx], add=True)
        pltpu.sync_copy(zmsg, acc.at[zsc], add=True)

        @pl.loop(0, CHUNK // 16)
        def _zclear(g):
            rows = iota16 + g * 16
            dv = scidx[pl.ds(g * 16, 16)]
            zc0 = lax.shift_left(jnp.bitwise_and(dv, 15), 3)
            for h in range(HH):
                plsc.store_scatter(zmsg, [rows, zc0 + h], zero16)

    plsc.subcore_barrier()

    @pl.loop(0, NB_PER_SUB)
    def _norm(m):
        base = s * (NB_PER_SUB * NB_CHUNK) + m * NB_CHUNK
        zoff = base // 16
        zalign = jnp.bitwise_and(zoff, ~7)
        zdelta = jnp.bitwise_and(zoff, 7)
        pltpu.sync_copy(acc.at[pl.ds(base, NB_CHUNK)], kbuf)
        pltpu.sync_copy(acc.at[pl.ds(ZB + zalign, 8)], zstage)

        @pl.loop(0, NB_CHUNK)
        def _node(n):
            zrow = jnp.full((16,), zdelta + lax.shift_right_logical(n, 4),
                            jnp.int32)
            zc0 = lax.shift_left(jnp.bitwise_and(n, 15), 3)
            for h in range(HH):
                zcol = jnp.full((16,), zc0 + h, jnp.int32)
                zh = plsc.load_gather(zstage, [zrow, zcol])
                wv = kbuf[n, pl.ds(h * HEAD_DIM, 16)]
                qbuf[n, pl.ds(h * HEAD_DIM, 16)] = wv / (zh + 1e-6)

        pltpu.sync_copy(qbuf, out.at[c, pl.ds(base, NB_CHUNK)])


@jax.jit
def _run(ktab, qtab, vtab, srcg, dstg, dsts):
    mesh = plsc.VectorSubcoreMesh(core_axis_name="c", subcore_axis_name="s",
                                  num_cores=NC, num_subcores=NS)
    return pl.kernel(
        _sc_body,
        out_type=jax.ShapeDtypeStruct((NC, WV_ROWS, HW), jnp.float32),
        mesh=mesh,
        compiler_params=pltpu.CompilerParams(needs_layout_passes=False),
        scratch_types=[
            pltpu.VMEM_SHARED((ACC_ROWS, HW), jnp.float32),
            pltpu.VMEM((CHUNK, HW), jnp.float32),
            pltpu.VMEM((CHUNK, HW), jnp.float32),
            pltpu.VMEM((CHUNK, HW), jnp.float32),
            pltpu.VMEM((CHUNK, HW), jnp.float32),
            pltpu.VMEM((CHUNK,), jnp.int32),
            pltpu.VMEM((CHUNK,), jnp.int32),
            pltpu.VMEM((CHUNK,), jnp.int32),
            pltpu.VMEM((CHUNK,), jnp.int32),
            pltpu.VMEM((8, HW), jnp.float32),
            pltpu.SemaphoreType.DMA,
        ],
    )(ktab, qtab, vtab, srcg, dstg, dsts)


def kernel(q, k, v, edge_index):
    batch, node_num = q.shape[0], q.shape[1]

    def half_tab(x):
        return (x.reshape(NODES, NC, HW)
                 .transpose(1, 0, 2)
                 .reshape(NC * NODES, HW))

    ktab = half_tab(k)
    qtab = half_tab(q)
    vtab = half_tab(v)

    src = edge_index[0].astype(jnp.int32)
    dst = edge_index[1].astype(jnp.int32)
    pad = E_PAD - EDGES
    src_p = jnp.concatenate([src, jnp.zeros((pad,), jnp.int32)])
    dst_gp = jnp.concatenate([dst, jnp.zeros((pad,), jnp.int32)])
    dst_sp = jnp.concatenate([dst, jnp.full((pad,), NODES, jnp.int32)])
    srcg = jnp.stack([src_p, src_p + NODES]).reshape(NC, NS, N_CHUNKS, CHUNK)
    dstg = jnp.stack([dst_gp, dst_gp + NODES]).reshape(NC, NS, N_CHUNKS, CHUNK)
    dsts = dst_sp.reshape(NS, N_CHUNKS, CHUNK)

    out2 = _run(ktab, qtab, vtab, srcg, dstg, dsts)
    return out2[:, :NODES].transpose(1, 0, 2).reshape(batch, node_num, HIDDEN)


# speedup vs baseline: 15.4827x; 2.5264x over previous
import jax
import jax.numpy as jnp
from jax import lax
from jax.experimental import pallas as pl
from jax.experimental.pallas import tpu as pltpu
from jax.experimental.pallas import tpu_sc as plsc

NUM_HEADS = 16
HEAD_DIM = 16
HIDDEN = NUM_HEADS * HEAD_DIM
SCALE = float(HEAD_DIM) ** 0.5
NODES = 10000
EDGES = 160000

NC = 2
NS = 16
HH = NUM_HEADS // NC
HW = HH * HEAD_DIM
CHUNK = 64
E_PAD = 163840
N_CHUNKS = E_PAD // (NS * CHUNK)
WV_ROWS = 10240
ZB = WV_ROWS
Z_ROWS = 640
ACC_ROWS = WV_ROWS + Z_ROWS
NB_CHUNK = 64
NB_PER_SUB = 10


def _sc_body(ktab, qtab, vtab, srcg, dstg, dsts, out,
             acc, kbuf, qbuf, msg, zmsg, sidx, didx, scidx, zsc,
             zstage, sem):
    c = lax.axis_index("c")
    s = lax.axis_index("s")
    zero16 = jnp.zeros((16,), jnp.float32)
    iota16 = lax.iota(jnp.int32, 16)

    @pl.loop(0, CHUNK)
    def _zero_rows(r):
        for cb in range(HW // 16):
            msg[r, pl.ds(cb * 16, 16)] = zero16
            zmsg[r, pl.ds(cb * 16, 16)] = zero16

    @pl.loop(0, 11)
    def _zero_acc(m):
        t = m * NS + s
        @pl.when(t < ACC_ROWS // CHUNK)
        def _():
            pltpu.sync_copy(msg, acc.at[pl.ds(t * CHUNK, CHUNK)])

    plsc.subcore_barrier()

    @pl.loop(0, N_CHUNKS)
    def _chunk(j):
        pltpu.sync_copy(srcg.at[c, s, j], sidx)
        pltpu.sync_copy(dstg.at[c, s, j], didx)
        pltpu.sync_copy(dsts.at[s, j], scidx)
        cp_k = pltpu.async_copy(ktab.at[sidx], kbuf, sem)
        cp_q = pltpu.async_copy(qtab.at[didx], qbuf, sem)
        cp_v = pltpu.async_copy(vtab.at[sidx], msg, sem)
        for g in range(CHUNK // 16):
            dv = scidx[pl.ds(g * 16, 16)]
            zsc[pl.ds(g * 16, 16)] = ZB + lax.shift_right_logical(dv, 4)
        cp_k.wait()
        cp_q.wait()
        cp_v.wait()

        @pl.loop(0, CHUNK // 16)
        def _group(g):
            rows = iota16 + g * 16
            dv = scidx[pl.ds(g * 16, 16)]
            zc0 = lax.shift_left(jnp.bitwise_and(dv, 15), 3)
            for h in range(HH):
                dot = zero16
                for d in range(HEAD_DIM):
                    col = h * HEAD_DIM + jnp.bitwise_and(d + iota16, 15)
                    kv = plsc.load_gather(kbuf, [rows, col])
                    qv = plsc.load_gather(qbuf, [rows, col])
                    dot = dot + kv * qv
                sc = dot * (1.0 / SCALE)
                sc = jnp.minimum(jnp.maximum(sc, -5.0), 5.0)
                es = jnp.exp(sc)
                plsc.store_scatter(zmsg, [rows, zc0 + h], es)
                for d in range(HEAD_DIM):
                    col = h * HEAD_DIM + jnp.bitwise_and(d + iota16, 15)
                    vv = plsc.load_gather(msg, [rows, col])
                    plsc.store_scatter(msg, [rows, col], vv * es)

        pltpu.sync_copy(msg, acc.at[scidx], add=True)
        pltpu.sync_copy(zmsg, acc.at[zsc], add=True)

        @pl.loop(0, CHUNK // 16)
        def _zclear(g):
            rows = iota16 + g * 16
            dv = scidx[pl.ds(g * 16, 16)]
            zc0 = lax.shift_left(jnp.bitwise_and(dv, 15), 3)
            for h in range(HH):
                plsc.store_scatter(zmsg, [rows, zc0 + h], zero16)

    plsc.subcore_barrier()

    @pl.loop(0, NB_PER_SUB)
    def _norm(m):
        base = s * (NB_PER_SUB * NB_CHUNK) + m * NB_CHUNK
        zoff = base // 16
        zalign = jnp.bitwise_and(zoff, ~7)
        zdelta = jnp.bitwise_and(zoff, 7)
        pltpu.sync_copy(acc.at[pl.ds(base, NB_CHUNK)], kbuf)
        pltpu.sync_copy(acc.at[pl.ds(ZB + zalign, 8)], zstage)

        @pl.loop(0, NB_CHUNK)
        def _node(n):
            zrow = jnp.full((16,), zdelta + lax.shift_right_logical(n, 4),
                            jnp.int32)
            zc0 = lax.shift_left(jnp.bitwise_and(n, 15), 3)
            for h in range(HH):
                zcol = jnp.full((16,), zc0 + h, jnp.int32)
                zh = plsc.load_gather(zstage, [zrow, zcol])
                wv = kbuf[n, pl.ds(h * HEAD_DIM, 16)]
                qbuf[n, pl.ds(h * HEAD_DIM, 16)] = wv / (zh + 1e-6)

        pltpu.sync_copy(qbuf, out.at[c, pl.ds(base, NB_CHUNK)])


@jax.jit
def _run(ktab, qtab, vtab, srcg, dstg, dsts):
    mesh = plsc.VectorSubcoreMesh(core_axis_name="c", subcore_axis_name="s",
                                  num_cores=NC, num_subcores=NS)
    return pl.kernel(
        _sc_body,
        out_type=jax.ShapeDtypeStruct((NC, WV_ROWS, HW), jnp.float32),
        mesh=mesh,
        compiler_params=pltpu.CompilerParams(needs_layout_passes=False),
        scratch_types=[
            pltpu.VMEM_SHARED((ACC_ROWS, HW), jnp.float32),
            pltpu.VMEM((CHUNK, HW), jnp.float32),
            pltpu.VMEM((CHUNK, HW), jnp.float32),
            pltpu.VMEM((CHUNK, HW), jnp.float32),
            pltpu.VMEM((CHUNK, HW), jnp.float32),
            pltpu.VMEM((CHUNK,), jnp.int32),
            pltpu.VMEM((CHUNK,), jnp.int32),
            pltpu.VMEM((CHUNK,), jnp.int32),
            pltpu.VMEM((CHUNK,), jnp.int32),
            pltpu.VMEM((8, HW), jnp.float32),
            pltpu.SemaphoreType.DMA,
        ],
    )(ktab, qtab, vtab, srcg, dstg, dsts)


def kernel(q, k, v, edge_index):
    batch, node_num = q.shape[0], q.shape[1]

    def half_tab(x):
        return (x.reshape(NODES, NC, HW)
                 .transpose(1, 0, 2)
                 .reshape(NC * NODES, HW))

    ktab = half_tab(k)
    qtab = half_tab(q)
    vtab = half_tab(v)

    src = edge_index[0].astype(jnp.int32)
    dst = edge_index[1].astype(jnp.int32)
    pad = E_PAD - EDGES
    src_p = jnp.concatenate([src, jnp.zeros((pad,), jnp.int32)])
    dst_gp = jnp.concatenate([dst, jnp.zeros((pad,), jnp.int32)])
    dst_sp = jnp.concatenate([dst, jnp.full((pad,), NODES, jnp.int32)])
    srcg = jnp.stack([src_p, src_p + NODES]).reshape(NC, NS, N_CHUNKS, CHUNK)
    dstg = jnp.stack([dst_gp, dst_gp + NODES]).reshape(NC, NS, N_CHUNKS, CHUNK)
    dsts = dst_sp.reshape(NS, N_CHUNKS, CHUNK)

    out2 = _run(ktab, qtab, vtab, srcg, dstg, dsts)
    return out2[:, :NODES].transpose(1, 0, 2).reshape(batch, node_num, HIDDEN)


# speedup vs baseline: 19.5106x; 1.2601x over previous
import jax
import jax.numpy as jnp
from jax import lax
from jax.experimental import pallas as pl
from jax.experimental.pallas import tpu as pltpu
from jax.experimental.pallas import tpu_sc as plsc

NUM_HEADS = 16
HEAD_DIM = 16
HIDDEN = NUM_HEADS * HEAD_DIM
SCALE = float(HEAD_DIM) ** 0.5
NODES = 10000
EDGES = 160000

NC = 2
NS = 16
HH = NUM_HEADS // NC
HW = HH * HEAD_DIM
CHUNK = 48
N_CHUNKS = 214
E_PAD = NS * N_CHUNKS * CHUNK
WV_ROWS = 10080
ZB = WV_ROWS
ACC_ROWS = 10752
GROUPS = CHUNK // 16


def _sc_body(ktab, qtab, vtab, srcg, dstg, dsts, out,
             acc, kbuf, qbuf, msg, zmsg, sidx, didx, scidx, zsc,
             semi, semk, semq, semv, semsc):
    c = lax.axis_index("c")
    s = lax.axis_index("s")
    zero16 = jnp.zeros((16,), jnp.float32)
    iota16 = lax.iota(jnp.int32, 16)

    @pl.loop(0, CHUNK)
    def _zero_rows(r):
        for cb in range(HW // 16):
            msg[r, pl.ds(cb * 16, 16)] = zero16
            zmsg[r, pl.ds(cb * 16, 16)] = zero16

    @pl.loop(0, ACC_ROWS // (NS * CHUNK))
    def _zero_acc(m):
        t = m * NS + s
        pltpu.sync_copy(msg, acc.at[pl.ds(t * CHUNK, CHUNK)])

    plsc.subcore_barrier()

    pltpu.sync_copy(srcg.at[c, s, 0], sidx.at[0, 0])
    pltpu.sync_copy(dstg.at[c, s, 0], didx.at[0, 0])
    pltpu.sync_copy(dsts.at[s, 0], scidx.at[0, 0])
    pltpu.async_copy(ktab.at[sidx.at[0, 0]], kbuf.at[0], semk)
    pltpu.async_copy(qtab.at[didx.at[0, 0]], qbuf.at[0], semq)
    pltpu.async_copy(vtab.at[sidx.at[0, 0]], msg, semv)

    @pl.loop(0, N_CHUNKS)
    def _chunk(j):
        p = jnp.bitwise_and(j, 1)
        pn = 1 - p
        jn = jnp.minimum(j + 1, N_CHUNKS - 1)

        pltpu.async_copy(srcg.at[c, s, jn], sidx.at[pn, 0], semi)
        pltpu.async_copy(dstg.at[c, s, jn], didx.at[pn, 0], semi)
        pltpu.async_copy(dsts.at[s, jn], scidx.at[pn, 0], semi)

        for g in range(GROUPS):
            dv = scidx[p, 0, pl.ds(g * 16, 16)]
            zsc[p, 0, pl.ds(g * 16, 16)] = ZB + lax.shift_right_logical(dv, 4)

        pltpu.make_async_copy(ktab.at[sidx.at[p, 0]], kbuf.at[p], semk).wait()
        pltpu.make_async_copy(qtab.at[didx.at[p, 0]], qbuf.at[p], semq).wait()

        @pl.loop(0, GROUPS)
        def _score(g):
            rows = iota16 + g * 16
            dv = scidx[p, 0, pl.ds(g * 16, 16)]
            zc0 = lax.shift_left(jnp.bitwise_and(dv, 15), 3)
            pv = jnp.full((16,), p, jnp.int32)
            for h in range(HH):
                dot = zero16
                for d in range(HEAD_DIM):
                    col = h * HEAD_DIM + jnp.bitwise_and(d + iota16, 15)
                    kv = plsc.load_gather(kbuf, [pv, rows, col])
                    qv = plsc.load_gather(qbuf, [pv, rows, col])
                    dot = dot + kv * qv
                sc = dot * (1.0 / SCALE)
                sc = jnp.minimum(jnp.maximum(sc, -5.0), 5.0)
                es = jnp.exp(sc)
                plsc.store_scatter(zmsg, [rows, zc0 + h], es)

        pltpu.make_async_copy(vtab.at[sidx.at[p, 0]], msg, semv).wait()

        @pl.loop(0, GROUPS)
        def _scale(g):
            rows = iota16 + g * 16
            dv = scidx[p, 0, pl.ds(g * 16, 16)]
            zc0 = lax.shift_left(jnp.bitwise_and(dv, 15), 3)
            for h in range(HH):
                es = plsc.load_gather(zmsg, [rows, zc0 + h])
                for d in range(HEAD_DIM):
                    col = h * HEAD_DIM + jnp.bitwise_and(d + iota16, 15)
                    vv = plsc.load_gather(msg, [rows, col])
                    plsc.store_scatter(msg, [rows, col], vv * es)

        pltpu.async_copy(msg, acc.at[scidx.at[p, 0]], semsc, add=True)
        pltpu.async_copy(zmsg, acc.at[zsc.at[p, 0]], semsc, add=True)

        pltpu.make_async_copy(srcg.at[c, s, jn], sidx.at[pn, 0], semi).wait()
        pltpu.make_async_copy(dstg.at[c, s, jn], didx.at[pn, 0], semi).wait()
        pltpu.make_async_copy(dsts.at[s, jn], scidx.at[pn, 0], semi).wait()
        pltpu.async_copy(ktab.at[sidx.at[pn, 0]], kbuf.at[pn], semk)
        pltpu.async_copy(qtab.at[didx.at[pn, 0]], qbuf.at[pn], semq)

        pltpu.make_async_copy(msg, acc.at[scidx.at[p, 0]], semsc).wait()
        pltpu.make_async_copy(zmsg, acc.at[zsc.at[p, 0]], semsc).wait()

        @pl.loop(0, GROUPS)
        def _zclear(g):
            rows = iota16 + g * 16
            dv = scidx[p, 0, pl.ds(g * 16, 16)]
            zc0 = lax.shift_left(jnp.bitwise_and(dv, 15), 3)
            for h in range(HH):
                plsc.store_scatter(zmsg, [rows, zc0 + h], zero16)

        pltpu.async_copy(vtab.at[sidx.at[pn, 0]], msg, semv)

    lastp = N_CHUNKS % 2
    pltpu.make_async_copy(ktab.at[sidx.at[lastp, 0]], kbuf.at[lastp],
                          semk).wait()
    pltpu.make_async_copy(qtab.at[didx.at[lastp, 0]], qbuf.at[lastp],
                          semq).wait()
    pltpu.make_async_copy(vtab.at[sidx.at[lastp, 0]], msg, semv).wait()

    plsc.subcore_barrier()

    @pl.loop(0, 14)
    def _norm(m):
        t = m * NS + s

        @pl.when(t < WV_ROWS // CHUNK)
        def _():
            base = t * CHUNK
            zoff = t * GROUPS
            zalign = jnp.bitwise_and(zoff, ~7)
            zdelta = zoff - zalign
            pltpu.sync_copy(acc.at[pl.ds(base, CHUNK)], kbuf.at[0])
            pltpu.sync_copy(acc.at[pl.ds(ZB + zalign, 16)],
                            kbuf.at[1, pl.ds(0, 16)])

            @pl.loop(0, CHUNK)
            def _node(n):
                zrow = jnp.full((16,),
                                zdelta + lax.shift_right_logical(n, 4),
                                jnp.int32)
                zc0 = lax.shift_left(jnp.bitwise_and(n, 15), 3)
                one = jnp.full((16,), 1, jnp.int32)
                for h in range(HH):
                    zcol = jnp.full((16,), zc0 + h, jnp.int32)
                    zh = plsc.load_gather(kbuf, [one, zrow, zcol])
                    wv = kbuf[0, n, pl.ds(h * HEAD_DIM, 16)]
                    qbuf[0, n, pl.ds(h * HEAD_DIM, 16)] = wv / (zh + 1e-6)

            pltpu.sync_copy(qbuf.at[0], out.at[c, pl.ds(base, CHUNK)])


@jax.jit
def _run(ktab, qtab, vtab, srcg, dstg, dsts):
    mesh = plsc.VectorSubcoreMesh(core_axis_name="c", subcore_axis_name="s",
                                  num_cores=NC, num_subcores=NS)
    return pl.kernel(
        _sc_body,
        out_type=jax.ShapeDtypeStruct((NC, WV_ROWS, HW), jnp.float32),
        mesh=mesh,
        compiler_params=pltpu.CompilerParams(needs_layout_passes=False),
        scratch_types=[
            pltpu.VMEM_SHARED((ACC_ROWS, HW), jnp.float32),
            pltpu.VMEM((2, CHUNK, HW), jnp.float32),
            pltpu.VMEM((2, CHUNK, HW), jnp.float32),
            pltpu.VMEM((CHUNK, HW), jnp.float32),
            pltpu.VMEM((CHUNK, HW), jnp.float32),
            pltpu.VMEM((2, 1, CHUNK), jnp.int32),
            pltpu.VMEM((2, 1, CHUNK), jnp.int32),
            pltpu.VMEM((2, 1, CHUNK), jnp.int32),
            pltpu.VMEM((2, 1, CHUNK), jnp.int32),
            pltpu.SemaphoreType.DMA,
            pltpu.SemaphoreType.DMA,
            pltpu.SemaphoreType.DMA,
            pltpu.SemaphoreType.DMA,
            pltpu.SemaphoreType.DMA,
        ],
    )(ktab, qtab, vtab, srcg, dstg, dsts)


def kernel(q, k, v, edge_index):
    batch, node_num = q.shape[0], q.shape[1]

    def half_tab(x):
        return (x.reshape(NODES, NC, HW)
                 .transpose(1, 0, 2)
                 .reshape(NC * NODES, HW))

    ktab = half_tab(k)
    qtab = half_tab(q)
    vtab = half_tab(v)

    src = edge_index[0].astype(jnp.int32)
    dst = edge_index[1].astype(jnp.int32)
    pad = E_PAD - EDGES
    src_p = jnp.concatenate([src, jnp.zeros((pad,), jnp.int32)])
    dst_gp = jnp.concatenate([dst, jnp.zeros((pad,), jnp.int32)])
    dst_sp = jnp.concatenate([dst, jnp.full((pad,), NODES, jnp.int32)])
    srcg = jnp.stack([src_p, src_p + NODES]).reshape(NC, NS, N_CHUNKS, CHUNK)
    dstg = jnp.stack([dst_gp, dst_gp + NODES]).reshape(NC, NS, N_CHUNKS, CHUNK)
    dsts = dst_sp.reshape(NS, N_CHUNKS, CHUNK)

    out2 = _run(ktab, qtab, vtab, srcg, dstg, dsts)
    return out2[:, :NODES].transpose(1, 0, 2).reshape(batch, node_num, HIDDEN)
